# baseline (speedup 1.0000x reference)
import functools

import jax
import jax.numpy as jnp
from jax import lax
from jax.experimental import pallas as pl
from jax.experimental.pallas import tpu as pltpu
from jax.experimental.pallas import tpu_sc as plsc

NC = 2
NS = 16
L = 16
NW = NC * NS

B = 128
V = 100000
RG = B // L

SCC = 49152
CLS_W = SCC // NW
CCH = 384
NCHK = CLS_W // CCH

TBW = 4096
TC_OFF = SCC // TBW
TC_STEPS = (V + TBW - 1) // TBW - TC_OFF

NEG_INF = float("-inf")
BIG_I = 2**30


def _sc_body(xt_hbm, mx_hbm, ix_hbm, bufa, bufb, mx_v, ix_v, sem0, sem1):
    c = lax.axis_index("c")
    s = lax.axis_index("s")
    wid = s * NC + c
    cls0 = wid * CLS_W

    bufs = (bufa, bufb)
    sems = (sem0, sem1)

    def src(t):
        return xt_hbm.at[pl.ds(cls0 + t * CCH, CCH), :]

    pend = [None, None]
    pend[0] = pltpu.async_copy(src(0), bufs[0], sems[0])

    negv = jnp.full((L,), NEG_INF, jnp.float32)
    zeroi = jnp.zeros((L,), jnp.int32)
    ams = [negv] * RG
    ais = [zeroi] * RG

    for t in range(NCHK):
        if t + 1 < NCHK:
            pend[(t + 1) % 2] = pltpu.async_copy(
                src(t + 1), bufs[(t + 1) % 2], sems[(t + 1) % 2]
            )
        pend[t % 2].wait()
        buf = bufs[t % 2]
        cb = cls0 + t * CCH

        def step(j, carry, buf=buf, cb=cb):
            accs = list(carry)
            idxv = jnp.broadcast_to(cb + j, (L,)).astype(jnp.int32)
            for r in range(RG):
                am = accs[2 * r]
                ai = accs[2 * r + 1]
                v = buf[j, pl.ds(r * L, L)]
                gt = v > am
                accs[2 * r] = jnp.where(gt, v, am)
                accs[2 * r + 1] = jnp.where(gt, idxv, ai)
            return tuple(accs)

        carry = tuple(x for pair in zip(ams, ais) for x in pair)
        carry = lax.fori_loop(0, CCH, step, carry, unroll=2)
        ams = list(carry[0::2])
        ais = list(carry[1::2])

    for r in range(RG):
        mx_v[pl.ds(r * L, L)] = ams[r]
        ix_v[pl.ds(r * L, L)] = ais[r]
    pltpu.sync_copy(mx_v, mx_hbm.at[wid])
    pltpu.sync_copy(ix_v, ix_hbm.at[wid])


@functools.cache
def _sc_rowmax():
    return functools.partial(
        pl.kernel,
        out_type=(
            jax.ShapeDtypeStruct((NW, B), jnp.float32),
            jax.ShapeDtypeStruct((NW, B), jnp.int32),
        ),
        mesh=plsc.VectorSubcoreMesh(
            core_axis_name="c", subcore_axis_name="s", num_cores=NC, num_subcores=NS
        ),
        scratch_types=[
            pltpu.VMEM((CCH, B), jnp.float32),
            pltpu.VMEM((CCH, B), jnp.float32),
            pltpu.VMEM((B,), jnp.float32),
            pltpu.VMEM((B,), jnp.int32),
            pltpu.SemaphoreType.DMA,
            pltpu.SemaphoreType.DMA,
        ],
    )(_sc_body)


def _tc_rowmax_body(x_ref, mx_ref, ix_ref, am_ref, ai_ref):
    i = pl.program_id(0)

    @pl.when(i == 0)
    def _init():
        am_ref[...] = jnp.full((8, B), NEG_INF, jnp.float32)
        ai_ref[...] = jnp.zeros((8, B), jnp.int32)

    NJ = TBW // 8

    def fold(masked):
        am = am_ref[...]
        ai = ai_ref[...]
        octet = jnp.broadcast_to(i * NJ, (8, B)).astype(jnp.int32)
        one = jnp.ones((8, B), jnp.int32)
        for j in range(NJ):
            v = x_ref[j * 8 : (j + 1) * 8, :]
            if masked:
                cls = (TC_OFF + i) * TBW + j * 8 + jax.lax.broadcasted_iota(
                    jnp.int32, (8, B), 0
                )
                v = jnp.where(cls < V, v, NEG_INF)
            gt = v > am
            am = jnp.maximum(am, v)
            ai = jnp.where(gt, octet, ai)
            octet = octet + one
        am_ref[...] = am
        ai_ref[...] = ai

    @pl.when(i < TC_STEPS - 1)
    def _hot():
        fold(masked=False)

    @pl.when(i == TC_STEPS - 1)
    def _last():
        fold(masked=True)
        a = am_ref[...]
        sub = jax.lax.broadcasted_iota(jnp.int32, (8, B), 0)
        cls = (ai_ref[...] * 8 + sub) + SCC
        m = jnp.max(a, axis=0, keepdims=True)
        cand = jnp.where(a == m, cls, BIG_I)
        mx_ref[...] = m
        ix_ref[...] = jnp.min(cand, axis=0, keepdims=True)


@functools.cache
def _tc_rowmax():
    return pl.pallas_call(
        _tc_rowmax_body,
        grid=(TC_STEPS,),
        in_specs=[pl.BlockSpec((TBW, B), lambda i: (TC_OFF + i, 0))],
        out_specs=[
            pl.BlockSpec((1, B), lambda i: (0, 0)),
            pl.BlockSpec((1, B), lambda i: (0, 0)),
        ],
        out_shape=(
            jax.ShapeDtypeStruct((1, B), jnp.float32),
            jax.ShapeDtypeStruct((1, B), jnp.int32),
        ),
        scratch_shapes=[
            pltpu.VMEM((8, B), jnp.float32),
            pltpu.VMEM((8, B), jnp.int32),
        ],
    )


def _finalize_body(scm_ref, sci_ref, tcm_ref, tci_ref, t_ref, o_ref):
    scm = scm_ref[...]
    sci = sci_ref[...]
    tcm = tcm_ref[...]
    tci = tci_ref[...]
    bm = jnp.maximum(jnp.max(scm, axis=0, keepdims=True), tcm)
    c_sc = jnp.min(jnp.where(scm == bm, sci, BIG_I), axis=0, keepdims=True)
    c_tc = jnp.where(tcm == bm, tci, BIG_I)
    best = jnp.minimum(c_sc, c_tc)
    ok = (best == t_ref[...].reshape(1, B)).astype(jnp.float32)
    o_ref[0] = jnp.sum(ok) * (100.0 / B)


_finalize = pl.pallas_call(
    _finalize_body,
    out_shape=jax.ShapeDtypeStruct((1,), jnp.float32),
    in_specs=[pl.BlockSpec(memory_space=pltpu.VMEM)] * 5,
    out_specs=pl.BlockSpec(memory_space=pltpu.SMEM),
)


@jax.jit
def kernel(output, target):
    xt = output.T
    mx, ix = _sc_rowmax()(xt)
    tm, ti = _tc_rowmax()(xt)
    return _finalize(mx, ix, tm, ti, target)
